# baseline (speedup 1.0000x reference)
import jax
import jax.numpy as jnp
from jax.experimental import pallas as pl
from jax.experimental.pallas import tpu as pltpu

N = 16384
C = 1000
KEEP = N // 2
R = 4096
NB = N // R


def _ce_bottomk_kernel(x_ref, lab_ref, out_ref, loss_ref):
    _MSB = jnp.int32(-2147483648)
    i = pl.program_id(0)
    x = x_ref[...]
    lab = lab_ref[0]
    m = jnp.max(x, axis=0, keepdims=True)
    s = jnp.sum(jnp.exp(x - m), axis=0, keepdims=True)
    rows = jax.lax.broadcasted_iota(jnp.int32, (C, R), 0)
    picked = jnp.sum(jnp.where(rows == lab, x, 0.0), axis=0, keepdims=True)
    loss = m + jnp.log(s) - picked
    loss_ref[i, :] = loss[0]

    @pl.when(i == NB - 1)
    def _select():
        v = loss_ref[...]
        b = jax.lax.bitcast_convert_type(v, jnp.int32)
        ukey = jax.lax.bitcast_convert_type(b ^ ((b >> 31) | _MSB), jnp.uint32)

        def body(r, lo):
            mid = lo | (jnp.uint32(1) << jnp.uint32(31 - r))
            c = jnp.sum(jnp.where(ukey < mid, 1, 0))
            return jnp.where(c < KEEP, mid, lo)

        kth = jax.lax.fori_loop(0, 32, body, jnp.uint32(0))
        below = ukey < kth
        c_lt = jnp.sum(jnp.where(below, 1, 0))
        s_lt = jnp.sum(jnp.where(below, v, 0.0))
        kb = jax.lax.bitcast_convert_type(kth, jnp.int32)
        kth_f = jax.lax.bitcast_convert_type(
            jnp.where(kb < 0, kb ^ _MSB, ~kb), jnp.float32)
        total = s_lt + (KEEP - c_lt).astype(jnp.float32) * kth_f
        out_ref[0, 0] = total / KEEP


def kernel(outputs, labels):
    xT = outputs.T
    labs = labels.astype(jnp.int32).reshape(NB, 1, R)
    out = pl.pallas_call(
        _ce_bottomk_kernel,
        grid=(NB,),
        in_specs=[
            pl.BlockSpec((C, R), lambda i: (0, i)),
            pl.BlockSpec((1, 1, R), lambda i: (i, 0, 0)),
        ],
        out_specs=pl.BlockSpec(memory_space=pltpu.SMEM),
        out_shape=jax.ShapeDtypeStruct((1, 1), jnp.float32),
        scratch_shapes=[pltpu.VMEM((NB, R), jnp.float32)],
        compiler_params=pltpu.CompilerParams(
            dimension_semantics=("arbitrary",)),
    )(xT, labs)
    return out[0, 0]


# speedup vs baseline: 1.0797x; 1.0797x over previous
import jax
import jax.numpy as jnp
from jax.experimental import pallas as pl
from jax.experimental.pallas import tpu as pltpu

N = 16384
C = 1000
KEEP = N // 2
R = 2048
NB = N // R


def _ce_bottomk_kernel(x_ref, lab_ref, out_ref, loss_ref):
    _MSB = jnp.int32(-2147483648)
    i = pl.program_id(0)
    x = x_ref[...]
    lab = lab_ref[0]
    m = jnp.max(x, axis=0, keepdims=True)
    t = x - m
    s = jnp.sum(jnp.exp(t), axis=0, keepdims=True)
    rows = jax.lax.broadcasted_iota(jnp.int32, (C, R), 0)
    picked_t = jnp.sum(jnp.where(rows == lab, t, 0.0), axis=0, keepdims=True)
    loss = jnp.log(s) - picked_t
    loss_ref[i, :] = loss[0]

    @pl.when(i == NB - 1)
    def _select():
        v = loss_ref[...]
        b = jax.lax.bitcast_convert_type(v, jnp.int32)
        ukey = jax.lax.bitcast_convert_type(b ^ ((b >> 31) | _MSB), jnp.uint32)

        def body(r, lo):
            mid = lo | (jnp.uint32(1) << jnp.uint32(31 - r))
            c = jnp.sum(jnp.where(ukey < mid, 1, 0))
            return jnp.where(c < KEEP, mid, lo)

        kth = jax.lax.fori_loop(0, 32, body, jnp.uint32(0))
        below = ukey < kth
        c_lt = jnp.sum(jnp.where(below, 1, 0))
        s_lt = jnp.sum(jnp.where(below, v, 0.0))
        kb = jax.lax.bitcast_convert_type(kth, jnp.int32)
        kth_f = jax.lax.bitcast_convert_type(
            jnp.where(kb < 0, kb ^ _MSB, ~kb), jnp.float32)
        total = s_lt + (KEEP - c_lt).astype(jnp.float32) * kth_f
        out_ref[0, 0] = total / KEEP


def kernel(outputs, labels):
    xT = outputs.T
    labs = labels.astype(jnp.int32).reshape(NB, 1, R)
    out = pl.pallas_call(
        _ce_bottomk_kernel,
        grid=(NB,),
        in_specs=[
            pl.BlockSpec((C, R), lambda i: (0, i)),
            pl.BlockSpec((1, 1, R), lambda i: (i, 0, 0)),
        ],
        out_specs=pl.BlockSpec(memory_space=pltpu.SMEM),
        out_shape=jax.ShapeDtypeStruct((1, 1), jnp.float32),
        scratch_shapes=[pltpu.VMEM((NB, R), jnp.float32)],
        compiler_params=pltpu.CompilerParams(
            dimension_semantics=("arbitrary",)),
    )(xT, labs)
    return out[0, 0]
